# baseline (speedup 1.0000x reference)
import functools

import jax
import jax.numpy as jnp
from jax import lax
from jax.experimental import pallas as pl
from jax.experimental.pallas import tpu as pltpu
from jax.experimental.pallas import tpu_sc as plsc

_BANK = 1000001
_DIM = 64
_BATCH = 16384
_MOM = 0.9

_NC, _NS = 2, 16
_NW = _NC * _NS
_BPW = _BATCH // _NW
_RND = 256
_G = 16
_NG = _RND // _G

_mesh = plsc.VectorSubcoreMesh(core_axis_name="c", subcore_axis_name="s")
_sc_params = pltpu.CompilerParams(use_tc_tiling_on_sc=True)


@functools.partial(
    pl.kernel,
    out_type=jax.ShapeDtypeStruct((_BATCH, _DIM), jnp.float32),
    mesh=_mesh,
    compiler_params=_sc_params,
    scratch_types=[
        pltpu.VMEM((_BPW,), jnp.int32),
        pltpu.VMEM((_RND, _DIM), jnp.float32),
        pltpu.VMEM((_RND, _DIM), jnp.float32),
        pltpu.VMEM((_RND, _DIM), jnp.float32),
        pltpu.SemaphoreType.DMA,
    ],
)
def _sc_update(idx_hbm, data_hbm, bank_ref, avgs_hbm,
               idx_v, rows0_v, rows1_v, data_v, sem):
    w = lax.axis_index("s") * _NC + lax.axis_index("c")
    base = w * _BPW

    pltpu.sync_copy(idx_hbm.at[pl.ds(base, _BPW)], idx_v)

    def gather_into(rows_v, hb):
        def drain(g):
            pltpu.make_async_copy(bank_ref.at[pl.ds(0, _G)],
                                  rows_v.at[pl.ds(g * _G, _G)], sem).wait()

        def gbody(g, carry):
            v = idx_v[pl.ds(hb + g * _G, _G)]
            for j in range(_G):
                pltpu.async_copy(bank_ref.at[pl.ds(v[j], 1)],
                                 rows_v.at[pl.ds(g * _G + j, 1)], sem)

            @pl.when(g > 0)
            def _():
                drain(g - 1)
            return carry

        lax.fori_loop(0, _NG, gbody, 0)
        drain(_NG - 1)

    gather_into(rows0_v, 0)
    gather_into(rows1_v, _RND)

    pltpu.sync_copy(rows0_v, avgs_hbm.at[pl.ds(base, _RND)])
    pltpu.sync_copy(rows1_v, avgs_hbm.at[pl.ds(base + _RND, _RND)])

    for h, rows_v in enumerate((rows0_v, rows1_v)):
        pltpu.sync_copy(data_hbm.at[pl.ds(base + h * _RND, _RND)], data_v)

        def cbody(i, carry, rows_v=rows_v):
            for k in range(_DIM // 16):
                sl = pl.ds(k * 16, 16)
                rows_v[i, sl] = (rows_v[i, sl] * _MOM
                                 + data_v[i, sl] * (1.0 - _MOM))
            return carry

        lax.fori_loop(0, _RND, cbody, 0)

    for h, rows_v in enumerate((rows0_v, rows1_v)):
        def drain_s(g, rows_v=rows_v):
            pltpu.make_async_copy(rows_v.at[pl.ds(g * _G, _G)],
                                  bank_ref.at[pl.ds(0, _G)], sem).wait()

        def sbody(g, carry, rows_v=rows_v, hb=h * _RND):
            v = idx_v[pl.ds(hb + g * _G, _G)]
            for j in range(_G):
                pltpu.async_copy(rows_v.at[pl.ds(g * _G + j, 1)],
                                 bank_ref.at[pl.ds(v[j], 1)], sem)

            @pl.when(g > 0)
            def _():
                drain_s(g - 1)
            return carry

        lax.fori_loop(0, _NG, sbody, 0)
        drain_s(_NG - 1)


def _avgs_t_body(x_ref, o_ref):
    o_ref[...] = x_ref[...].T


def _avgs_transpose(avgs_rm):
    return pl.pallas_call(
        _avgs_t_body,
        grid=(16,),
        in_specs=[pl.BlockSpec((1024, _DIM), lambda i: (i, 0))],
        out_specs=pl.BlockSpec((_DIM, 1024), lambda i: (0, i)),
        out_shape=jax.ShapeDtypeStruct((_DIM, _BATCH), jnp.float32),
    )(avgs_rm)


def kernel(idx, data, memory):
    idx = idx.astype(jnp.int32)
    ar = jnp.arange(_BANK, dtype=jnp.int32)
    bank_ref = jax.new_ref(jnp.take(memory, ar, axis=0))
    avgs_rm = _sc_update(idx, data, bank_ref)
    return _avgs_transpose(avgs_rm).T, jnp.take(bank_ref[...], ar, axis=0)


# speedup vs baseline: 18.3234x; 18.3234x over previous
import functools

import jax
import jax.numpy as jnp
from jax import lax
from jax.experimental import pallas as pl
from jax.experimental.pallas import tpu as pltpu
from jax.experimental.pallas import tpu_sc as plsc

_BANK = 1000001
_DIM = 64
_BATCH = 16384
_MOM = 0.9

_NC, _NS = 2, 16
_NW = _NC * _NS
_BPW = _BATCH // _NW
_RND = 256
_G = 16
_NG = _RND // _G

_mesh = plsc.VectorSubcoreMesh(core_axis_name="c", subcore_axis_name="s")
_sc_params = pltpu.CompilerParams(use_tc_tiling_on_sc=True)


@functools.partial(
    pl.kernel,
    out_type=jax.ShapeDtypeStruct((_BATCH, _DIM), jnp.float32),
    mesh=_mesh,
    compiler_params=_sc_params,
    scratch_types=[
        pltpu.VMEM((_BPW,), jnp.int32),
        pltpu.VMEM((_RND, _DIM), jnp.float32),
        pltpu.VMEM((_RND, _DIM), jnp.float32),
        pltpu.VMEM((_RND, _DIM), jnp.float32),
        pltpu.SemaphoreType.DMA,
    ],
)
def _sc_update(idx_hbm, data_hbm, bank_ref, avgs_hbm,
               idx_v, rows0_v, rows1_v, data_v, sem):
    w = lax.axis_index("s") * _NC + lax.axis_index("c")
    base = w * _BPW

    pltpu.sync_copy(idx_hbm.at[pl.ds(base, _BPW)], idx_v)

    def gather_into(rows_v, hb):
        def drain(g):
            pltpu.make_async_copy(bank_ref.at[pl.ds(0, _G)],
                                  rows_v.at[pl.ds(g * _G, _G)], sem).wait()

        def gbody(g, carry):
            v = idx_v[pl.ds(hb + g * _G, _G)]
            for j in range(_G):
                pltpu.async_copy(bank_ref.at[pl.ds(v[j], 1)],
                                 rows_v.at[pl.ds(g * _G + j, 1)], sem)

            @pl.when(g > 0)
            def _():
                drain(g - 1)
            return carry

        lax.fori_loop(0, _NG, gbody, 0)
        drain(_NG - 1)

    gather_into(rows0_v, 0)
    gather_into(rows1_v, _RND)

    pltpu.sync_copy(rows0_v, avgs_hbm.at[pl.ds(base, _RND)])
    pltpu.sync_copy(rows1_v, avgs_hbm.at[pl.ds(base + _RND, _RND)])

    for h, rows_v in enumerate((rows0_v, rows1_v)):
        pltpu.sync_copy(data_hbm.at[pl.ds(base + h * _RND, _RND)], data_v)

        def cbody(i, carry, rows_v=rows_v):
            for k in range(_DIM // 16):
                sl = pl.ds(k * 16, 16)
                rows_v[i, sl] = (rows_v[i, sl] * _MOM
                                 + data_v[i, sl] * (1.0 - _MOM))
            return carry

        lax.fori_loop(0, _RND, cbody, 0)

    for h, rows_v in enumerate((rows0_v, rows1_v)):
        def drain_s(g, rows_v=rows_v):
            pltpu.make_async_copy(rows_v.at[pl.ds(g * _G, _G)],
                                  bank_ref.at[pl.ds(0, _G)], sem).wait()

        def sbody(g, carry, rows_v=rows_v, hb=h * _RND):
            v = idx_v[pl.ds(hb + g * _G, _G)]
            for j in range(_G):
                pltpu.async_copy(rows_v.at[pl.ds(g * _G + j, 1)],
                                 bank_ref.at[pl.ds(v[j], 1)], sem)

            @pl.when(g > 0)
            def _():
                drain_s(g - 1)
            return carry

        lax.fori_loop(0, _NG, sbody, 0)
        drain_s(_NG - 1)


def _avgs_t_body(x_ref, o_ref):
    o_ref[...] = x_ref[...].T


def _avgs_transpose(avgs_rm):
    return pl.pallas_call(
        _avgs_t_body,
        grid=(16,),
        in_specs=[pl.BlockSpec((1024, _DIM), lambda i: (i, 0))],
        out_specs=pl.BlockSpec((_DIM, 1024), lambda i: (0, i)),
        out_shape=jax.ShapeDtypeStruct((_DIM, _BATCH), jnp.float32),
    )(avgs_rm)


def kernel(idx, data, memory):
    idx = idx.astype(jnp.int32)
    bank_ref = jax.new_ref(lax.optimization_barrier(memory))
    avgs_rm = _sc_update(idx, data, bank_ref)
    return _avgs_transpose(avgs_rm).T, lax.optimization_barrier(bank_ref[...])
